# baseline (speedup 1.0000x reference)
import jax
import jax.numpy as jnp
from jax import lax
from jax.experimental import pallas as pl
from jax.experimental.pallas import tpu as pltpu
from jax.experimental.pallas import tpu_sc as plsc

_B = 16
_SEQ = 20
_N = 32
_NP = _N // 2
_H = 64
_W = 64
_BT = _B * _SEQ
_PTS = _BT * _NP

_NC = 2
_NS = 16
_NW = _NC * _NS
_IMGS_PER_TILE = _BT // _NW
_PPT = _IMGS_PER_TILE * _NP


def _sc_body(xs_hbm, ys_hbm, rx_hbm, ry_hbm, ox_hbm, oy_hbm, idx_hbm,
             xsv, ysv, rxv, ryv, oxv, oyv, idxv):
    wid = lax.axis_index("s") * _NC + lax.axis_index("c")
    base = wid * _PPT

    pltpu.sync_copy(xs_hbm.at[pl.ds(base, _PPT)], xsv)
    pltpu.sync_copy(ys_hbm.at[pl.ds(base, _PPT)], ysv)
    pltpu.sync_copy(rx_hbm.at[pl.ds(base, _PPT)], rxv)
    pltpu.sync_copy(ry_hbm.at[pl.ds(base, _PPT)], ryv)
    pltpu.sync_copy(ox_hbm.at[pl.ds(base, _PPT)], oxv)
    pltpu.sync_copy(oy_hbm.at[pl.ds(base, _PPT)], oyv)

    for i in range(_IMGS_PER_TILE):
        s = i * _NP
        xs = xsv[pl.ds(s, 16)]
        ys = ysv[pl.ds(s, 16)]
        rx = rxv[pl.ds(s, 16)]
        ry = ryv[pl.ds(s, 16)]
        ox = oxv[pl.ds(s, 16)]
        oy = oyv[pl.ds(s, 16)]
        col = (xs / rx + ox).astype(jnp.int32)
        row = (ys / ry + oy).astype(jnp.int32)
        idxv[pl.ds(s, 16)] = row * _W + col

    pltpu.sync_copy(idxv, idx_hbm.at[pl.ds(base, _PPT)])


@jax.jit
def _sc_indices(xs, ys, rx, ry, ox, oy):
    mesh = plsc.VectorSubcoreMesh(core_axis_name="c", subcore_axis_name="s")
    return pl.kernel(
        _sc_body,
        out_type=jax.ShapeDtypeStruct((_PTS,), jnp.int32),
        mesh=mesh,
        scratch_types=[
            pltpu.VMEM((_PPT,), jnp.float32),
            pltpu.VMEM((_PPT,), jnp.float32),
            pltpu.VMEM((_PPT,), jnp.float32),
            pltpu.VMEM((_PPT,), jnp.float32),
            pltpu.VMEM((_PPT,), jnp.float32),
            pltpu.VMEM((_PPT,), jnp.float32),
            pltpu.VMEM((_PPT,), jnp.int32),
        ],
    )(xs, ys, rx, ry, ox, oy)


_BPB = 1
_NBLK = _B // _BPB


def _tc_body(idx_ref, out_ref):
    out_ref[...] = jnp.zeros((_BPB, _SEQ, _H, _NP, _W), jnp.float32)
    wio = lax.broadcasted_iota(jnp.int32, (1, 1, _W), 2)
    for bl in range(_BPB):
        for tl in range(_SEQ):
            for p in range(_NP):
                v = idx_ref[bl, tl, p]
                r = v // _W
                c = v - r * _W
                oh = jnp.where(wio == c, 1.0, 0.0).astype(jnp.float32)
                out_ref[bl, tl, pl.ds(r, 1), pl.ds(p, 1), :] = oh


@jax.jit
def _tc_raster(idx3):
    return pl.pallas_call(
        _tc_body,
        grid=(_NBLK,),
        in_specs=[pl.BlockSpec((_BPB, _SEQ, _NP), lambda i: (i, 0, 0),
                               memory_space=pltpu.SMEM)],
        out_specs=pl.BlockSpec((_BPB, _SEQ, _H, _NP, _W),
                               lambda i: (i, 0, 0, 0, 0)),
        out_shape=jax.ShapeDtypeStruct((_B, _SEQ, _H, _NP, _W), jnp.float32),
    )(idx3)


def kernel(x, resolution, origin):
    pts = x.reshape(_PTS, 2)
    xs = pts[:, 0]
    ys = pts[:, 1]
    rx = jnp.broadcast_to(resolution[:, :, None, 0], (_B, _SEQ, _NP)).reshape(-1)
    ry = jnp.broadcast_to(resolution[:, :, None, 1], (_B, _SEQ, _NP)).reshape(-1)
    ox = jnp.broadcast_to(origin[:, :, None, 0], (_B, _SEQ, _NP)).reshape(-1)
    oy = jnp.broadcast_to(origin[:, :, None, 1], (_B, _SEQ, _NP)).reshape(-1)
    idx = _sc_indices(xs, ys, rx, ry, ox, oy)
    out = _tc_raster(idx.reshape(_B, _SEQ, _NP))
    return jnp.swapaxes(out, 3, 4)


# speedup vs baseline: 1.0685x; 1.0685x over previous
import jax
import jax.numpy as jnp
from jax import lax
from jax.experimental import pallas as pl
from jax.experimental.pallas import tpu as pltpu
from jax.experimental.pallas import tpu_sc as plsc

_B = 16
_SEQ = 20
_N = 32
_NP = _N // 2
_H = 64
_W = 64
_BT = _B * _SEQ
_PTS = _BT * _NP

_NC = 2
_NS = 16
_NW = _NC * _NS
_IMGS_PER_TILE = _BT // _NW
_PPT = _IMGS_PER_TILE * _NP


def _sc_body(pk_hbm, idx_hbm, pkv, idxv):
    wid = lax.axis_index("s") * _NC + lax.axis_index("c")
    base = wid * _PPT

    pltpu.sync_copy(pk_hbm.at[pl.ds(wid * 6 * _PPT, 6 * _PPT)], pkv)

    for i in range(_IMGS_PER_TILE):
        s = i * _NP
        xs = pkv[pl.ds(0 * _PPT + s, 16)]
        ys = pkv[pl.ds(1 * _PPT + s, 16)]
        rx = pkv[pl.ds(2 * _PPT + s, 16)]
        ry = pkv[pl.ds(3 * _PPT + s, 16)]
        ox = pkv[pl.ds(4 * _PPT + s, 16)]
        oy = pkv[pl.ds(5 * _PPT + s, 16)]
        col = (xs / rx + ox).astype(jnp.int32)
        row = (ys / ry + oy).astype(jnp.int32)
        idxv[pl.ds(s, 16)] = row * _W + col

    pltpu.sync_copy(idxv, idx_hbm.at[pl.ds(base, _PPT)])


@jax.jit
def _sc_indices(packed):
    mesh = plsc.VectorSubcoreMesh(core_axis_name="c", subcore_axis_name="s")
    return pl.kernel(
        _sc_body,
        out_type=jax.ShapeDtypeStruct((_PTS,), jnp.int32),
        mesh=mesh,
        scratch_types=[
            pltpu.VMEM((6 * _PPT,), jnp.float32),
            pltpu.VMEM((_PPT,), jnp.int32),
        ],
    )(packed)


_BPB = 1
_NBLK = _B // _BPB


def _tc_body(idx_ref, out_ref):
    out_ref[...] = jnp.zeros((_BPB, _SEQ, _H, _NP, _W), jnp.float32)
    wio = lax.broadcasted_iota(jnp.int32, (1, 1, _W), 2)
    base = pl.program_id(0) * (_BPB * _SEQ * _NP)
    for bl in range(_BPB):
        for tl in range(_SEQ):
            for p in range(_NP):
                v = idx_ref[base + (bl * _SEQ + tl) * _NP + p]
                r = v // _W
                c = v - r * _W
                oh = jnp.where(wio == c, 1.0, 0.0).astype(jnp.float32)
                out_ref[bl, tl, pl.ds(r, 1), pl.ds(p, 1), :] = oh


@jax.jit
def _tc_raster(idx3):
    return pl.pallas_call(
        _tc_body,
        grid=(_NBLK,),
        in_specs=[pl.BlockSpec((_PTS,), lambda i: (0,),
                               memory_space=pltpu.SMEM)],
        out_specs=pl.BlockSpec((_BPB, _SEQ, _H, _NP, _W),
                               lambda i: (i, 0, 0, 0, 0)),
        out_shape=jax.ShapeDtypeStruct((_B, _SEQ, _H, _NP, _W), jnp.float32),
    )(idx3)


def kernel(x, resolution, origin):
    pts = x.reshape(_PTS, 2)
    xs = pts[:, 0]
    ys = pts[:, 1]
    rx = jnp.broadcast_to(resolution[:, :, None, 0], (_B, _SEQ, _NP)).reshape(-1)
    ry = jnp.broadcast_to(resolution[:, :, None, 1], (_B, _SEQ, _NP)).reshape(-1)
    ox = jnp.broadcast_to(origin[:, :, None, 0], (_B, _SEQ, _NP)).reshape(-1)
    oy = jnp.broadcast_to(origin[:, :, None, 1], (_B, _SEQ, _NP)).reshape(-1)
    packed = jnp.stack([xs, ys, rx, ry, ox, oy]).reshape(
        6, _NW, _PPT).transpose(1, 0, 2).reshape(-1)
    idx = _sc_indices(packed)
    out = _tc_raster(idx)
    return jnp.swapaxes(out, 3, 4)
